# baseline (speedup 1.0000x reference)
import functools
import math

import jax
import jax.numpy as jnp
from jax.experimental import pallas as pl
from jax.experimental.pallas import tpu as pltpu

S = 30.0
M = 1.5

_BR = 16


def _phi(v):
    c = (2.0 * v - 1.0) * jnp.sqrt((1.0 + v) * 0.5)
    return jnp.where(v < -0.5, -c - 2.0, c)


def _block_kernel(y_ref, x_ref, o_ref, *, C, row0):
    yb = y_ref[...]
    xb = x_ref[...]
    col = jax.lax.broadcasted_iota(jnp.int32, (_BR, C), 1)
    mask = col == yb
    val = jnp.sum(jnp.where(mask, xb, 0.0), axis=1, keepdims=True)
    special = S * _phi(val)
    o_ref[...] = jnp.where(mask, special, S * xb)


def _half(x, y2, C, row0, rows):
    return pl.pallas_call(
        functools.partial(_block_kernel, C=C, row0=row0),
        grid=(rows // _BR,),
        in_specs=[
            pl.BlockSpec((_BR, 1), lambda r: (r, 0)),
            pl.BlockSpec((_BR, C), lambda r: (r, 0)),
        ],
        out_specs=pl.BlockSpec((_BR, C), lambda r: (r, 0)),
        out_shape=jax.ShapeDtypeStruct((rows, C), jnp.float32),
        compiler_params=pltpu.CompilerParams(
            dimension_semantics=("parallel",),
        ),
    )(y2[row0:row0 + rows], x[row0:row0 + rows])


@jax.jit
def kernel(x, y):
    B, C = x.shape
    y2 = y.astype(jnp.int32).reshape(B, 1)
    h = B // 2
    top = _half(x, y2, C, 0, h)
    bot = _half(x, y2, C, h, h)
    return jnp.concatenate([top, bot], axis=0)
